# baseline (speedup 1.0000x reference)
import functools

import jax
import jax.numpy as jnp
from jax.experimental import pallas as pl
from jax.experimental.pallas import tpu as pltpu


def _body(x_ref, plo_ref, phi_ref, elo_ref, ehi_ref, ptab_ref, etab_ref,
          out_ref):
    vp = x_ref[0, 0:1, :]
    ve = x_ref[0, 1:2, :]
    oh_p = ((plo_ref[:, :] < vp) & (phi_ref[:, :] >= vp)).astype(jnp.bfloat16)
    oh_e = ((elo_ref[:, :] < ve) & (ehi_ref[:, :] >= ve)).astype(jnp.bfloat16)
    out_ref[0] = (
        jnp.dot(ptab_ref[:, :], oh_p, preferred_element_type=jnp.float32)
        + jnp.dot(etab_ref[:, :], oh_e, preferred_element_type=jnp.float32)
    )


@functools.partial(jax.jit, static_argnames=("interpret",))
def kernel(x, pitch_bins, energy_bins, pitch_embedding, energy_embedding,
           interpret=False):
    B, _, T = x.shape
    N, H = pitch_embedding.shape
    Tt = 512

    inf = jnp.array([jnp.inf], dtype=jnp.float32)
    p_lo = jnp.concatenate([-inf, pitch_bins])[:, None]
    p_hi = jnp.concatenate([pitch_bins, inf])[:, None]
    e_lo = jnp.concatenate([-inf, energy_bins])[:, None]
    e_hi = jnp.concatenate([energy_bins, inf])[:, None]
    ptab = pitch_embedding.T.astype(jnp.bfloat16)
    etab = energy_embedding.T.astype(jnp.bfloat16)

    grid = (B, T // Tt)
    return pl.pallas_call(
        _body,
        grid=grid,
        in_specs=[
            pl.BlockSpec((1, 2, Tt), lambda b, j: (b, 0, j)),
            pl.BlockSpec((N, 1), lambda b, j: (0, 0)),
            pl.BlockSpec((N, 1), lambda b, j: (0, 0)),
            pl.BlockSpec((N, 1), lambda b, j: (0, 0)),
            pl.BlockSpec((N, 1), lambda b, j: (0, 0)),
            pl.BlockSpec((H, N), lambda b, j: (0, 0)),
            pl.BlockSpec((H, N), lambda b, j: (0, 0)),
        ],
        out_specs=pl.BlockSpec((1, H, Tt), lambda b, j: (b, 0, j)),
        out_shape=jax.ShapeDtypeStruct((B, H, T), jnp.float32),
        compiler_params=pltpu.CompilerParams(
            dimension_semantics=("parallel", "parallel")),
        interpret=interpret,
    )(x, p_lo, p_hi, e_lo, e_hi, ptab, etab)


# speedup vs baseline: 1.5313x; 1.5313x over previous
import functools

import jax
import jax.numpy as jnp
from jax.experimental import pallas as pl
from jax.experimental.pallas import tpu as pltpu


def _body(x_ref, plo_ref, phi_ref, elo_ref, ehi_ref, ptab_ref, etab_ref,
          out_ref):
    vp = x_ref[0, 0:1, :]
    ve = x_ref[0, 1:2, :]
    oh_p = ((plo_ref[:, :] < vp) & (phi_ref[:, :] >= vp)).astype(jnp.bfloat16)
    oh_e = ((elo_ref[:, :] < ve) & (ehi_ref[:, :] >= ve)).astype(jnp.bfloat16)
    out_ref[0] = (
        jnp.dot(ptab_ref[:, :], oh_p, preferred_element_type=jnp.float32)
        + jnp.dot(etab_ref[:, :], oh_e, preferred_element_type=jnp.float32)
    )


@functools.partial(jax.jit, static_argnames=("interpret",))
def kernel(x, pitch_bins, energy_bins, pitch_embedding, energy_embedding,
           interpret=False):
    B, _, T = x.shape
    N, H = pitch_embedding.shape
    Tt = 1024

    inf = jnp.array([jnp.inf], dtype=jnp.float32)
    p_lo = jnp.concatenate([-inf, pitch_bins])[:, None]
    p_hi = jnp.concatenate([pitch_bins, inf])[:, None]
    e_lo = jnp.concatenate([-inf, energy_bins])[:, None]
    e_hi = jnp.concatenate([energy_bins, inf])[:, None]
    ptab = pitch_embedding.T.astype(jnp.bfloat16)
    etab = energy_embedding.T.astype(jnp.bfloat16)

    grid = (B, T // Tt)
    return pl.pallas_call(
        _body,
        grid=grid,
        in_specs=[
            pl.BlockSpec((1, 2, Tt), lambda b, j: (b, 0, j)),
            pl.BlockSpec((N, 1), lambda b, j: (0, 0)),
            pl.BlockSpec((N, 1), lambda b, j: (0, 0)),
            pl.BlockSpec((N, 1), lambda b, j: (0, 0)),
            pl.BlockSpec((N, 1), lambda b, j: (0, 0)),
            pl.BlockSpec((H, N), lambda b, j: (0, 0)),
            pl.BlockSpec((H, N), lambda b, j: (0, 0)),
        ],
        out_specs=pl.BlockSpec((1, H, Tt), lambda b, j: (b, 0, j)),
        out_shape=jax.ShapeDtypeStruct((B, H, T), jnp.float32),
        compiler_params=pltpu.CompilerParams(
            dimension_semantics=("parallel", "parallel")),
        interpret=interpret,
    )(x, p_lo, p_hi, e_lo, e_hi, ptab, etab)


# speedup vs baseline: 1.9850x; 1.2963x over previous
import functools

import jax
import jax.numpy as jnp
from jax.experimental import pallas as pl
from jax.experimental.pallas import tpu as pltpu


def _body(x_ref, plo_ref, phi_ref, elo_ref, ehi_ref, ptab_ref, etab_ref,
          out_ref):
    vp = x_ref[0, 0:1, :]
    ve = x_ref[0, 1:2, :]
    oh_p = ((plo_ref[:, :] < vp) & (phi_ref[:, :] >= vp)).astype(jnp.bfloat16)
    oh_e = ((elo_ref[:, :] < ve) & (ehi_ref[:, :] >= ve)).astype(jnp.bfloat16)
    out_ref[0] = (
        jnp.dot(ptab_ref[:, :], oh_p, preferred_element_type=jnp.float32)
        + jnp.dot(etab_ref[:, :], oh_e, preferred_element_type=jnp.float32)
    )


@functools.partial(jax.jit, static_argnames=("interpret",))
def kernel(x, pitch_bins, energy_bins, pitch_embedding, energy_embedding,
           interpret=False):
    B, _, T = x.shape
    N, H = pitch_embedding.shape
    Tt = 2048

    inf = jnp.array([jnp.inf], dtype=jnp.float32)
    p_lo = jnp.concatenate([-inf, pitch_bins])[:, None]
    p_hi = jnp.concatenate([pitch_bins, inf])[:, None]
    e_lo = jnp.concatenate([-inf, energy_bins])[:, None]
    e_hi = jnp.concatenate([energy_bins, inf])[:, None]
    ptab = pitch_embedding.T.astype(jnp.bfloat16)
    etab = energy_embedding.T.astype(jnp.bfloat16)

    grid = (B, T // Tt)
    return pl.pallas_call(
        _body,
        grid=grid,
        in_specs=[
            pl.BlockSpec((1, 2, Tt), lambda b, j: (b, 0, j)),
            pl.BlockSpec((N, 1), lambda b, j: (0, 0)),
            pl.BlockSpec((N, 1), lambda b, j: (0, 0)),
            pl.BlockSpec((N, 1), lambda b, j: (0, 0)),
            pl.BlockSpec((N, 1), lambda b, j: (0, 0)),
            pl.BlockSpec((H, N), lambda b, j: (0, 0)),
            pl.BlockSpec((H, N), lambda b, j: (0, 0)),
        ],
        out_specs=pl.BlockSpec((1, H, Tt), lambda b, j: (b, 0, j)),
        out_shape=jax.ShapeDtypeStruct((B, H, T), jnp.float32),
        compiler_params=pltpu.CompilerParams(
            dimension_semantics=("parallel", "parallel")),
        interpret=interpret,
    )(x, p_lo, p_hi, e_lo, e_hi, ptab, etab)


# speedup vs baseline: 2.1961x; 1.1064x over previous
import functools

import jax
import jax.numpy as jnp
from jax.experimental import pallas as pl
from jax.experimental.pallas import tpu as pltpu


def _body(x_ref, plo_ref, phi_ref, elo_ref, ehi_ref, ptab_ref, etab_ref,
          out_ref):
    nb = x_ref.shape[0]
    for i in range(nb):
        vp = x_ref[i, 0:1, :]
        ve = x_ref[i, 1:2, :]
        oh_p = ((plo_ref[:, :] < vp)
                & (phi_ref[:, :] >= vp)).astype(jnp.bfloat16)
        oh_e = ((elo_ref[:, :] < ve)
                & (ehi_ref[:, :] >= ve)).astype(jnp.bfloat16)
        out_ref[i] = (
            jnp.dot(ptab_ref[:, :], oh_p, preferred_element_type=jnp.float32)
            + jnp.dot(etab_ref[:, :], oh_e, preferred_element_type=jnp.float32)
        )


@functools.partial(jax.jit, static_argnames=("interpret",))
def kernel(x, pitch_bins, energy_bins, pitch_embedding, energy_embedding,
           interpret=False):
    B, _, T = x.shape
    N, H = pitch_embedding.shape
    Tt = 2048
    Bb = 2

    inf = jnp.array([jnp.inf], dtype=jnp.float32)
    p_lo = jnp.concatenate([-inf, pitch_bins])[:, None]
    p_hi = jnp.concatenate([pitch_bins, inf])[:, None]
    e_lo = jnp.concatenate([-inf, energy_bins])[:, None]
    e_hi = jnp.concatenate([energy_bins, inf])[:, None]
    ptab = pitch_embedding.T.astype(jnp.bfloat16)
    etab = energy_embedding.T.astype(jnp.bfloat16)

    grid = (B // Bb, T // Tt)
    return pl.pallas_call(
        _body,
        grid=grid,
        in_specs=[
            pl.BlockSpec((Bb, 2, Tt), lambda b, j: (b, 0, j)),
            pl.BlockSpec((N, 1), lambda b, j: (0, 0)),
            pl.BlockSpec((N, 1), lambda b, j: (0, 0)),
            pl.BlockSpec((N, 1), lambda b, j: (0, 0)),
            pl.BlockSpec((N, 1), lambda b, j: (0, 0)),
            pl.BlockSpec((H, N), lambda b, j: (0, 0)),
            pl.BlockSpec((H, N), lambda b, j: (0, 0)),
        ],
        out_specs=pl.BlockSpec((Bb, H, Tt), lambda b, j: (b, 0, j)),
        out_shape=jax.ShapeDtypeStruct((B, H, T), jnp.float32),
        compiler_params=pltpu.CompilerParams(
            dimension_semantics=("parallel", "parallel")),
        interpret=interpret,
    )(x, p_lo, p_hi, e_lo, e_hi, ptab, etab)


# speedup vs baseline: 2.2654x; 1.0315x over previous
import functools

import jax
import jax.numpy as jnp
from jax.experimental import pallas as pl
from jax.experimental.pallas import tpu as pltpu


def _body(x_ref, plo_ref, phi_ref, elo_ref, ehi_ref, ptab_ref, etab_ref,
          out_ref):
    nb = x_ref.shape[0]
    for i in range(nb):
        vp = x_ref[i, 0:1, :]
        ve = x_ref[i, 1:2, :]
        oh_p = ((plo_ref[:, :] < vp)
                & (phi_ref[:, :] >= vp)).astype(jnp.bfloat16)
        oh_e = ((elo_ref[:, :] < ve)
                & (ehi_ref[:, :] >= ve)).astype(jnp.bfloat16)
        out_ref[i] = (
            jnp.dot(ptab_ref[:, :], oh_p, preferred_element_type=jnp.float32)
            + jnp.dot(etab_ref[:, :], oh_e, preferred_element_type=jnp.float32)
        )


@functools.partial(jax.jit, static_argnames=("interpret",))
def kernel(x, pitch_bins, energy_bins, pitch_embedding, energy_embedding,
           interpret=False):
    B, _, T = x.shape
    N, H = pitch_embedding.shape
    Tt = 2048
    Bb = 4

    inf = jnp.array([jnp.inf], dtype=jnp.float32)
    p_lo = jnp.concatenate([-inf, pitch_bins])[:, None]
    p_hi = jnp.concatenate([pitch_bins, inf])[:, None]
    e_lo = jnp.concatenate([-inf, energy_bins])[:, None]
    e_hi = jnp.concatenate([energy_bins, inf])[:, None]
    ptab = pitch_embedding.T.astype(jnp.bfloat16)
    etab = energy_embedding.T.astype(jnp.bfloat16)

    grid = (B // Bb, T // Tt)
    return pl.pallas_call(
        _body,
        grid=grid,
        in_specs=[
            pl.BlockSpec((Bb, 2, Tt), lambda b, j: (b, 0, j)),
            pl.BlockSpec((N, 1), lambda b, j: (0, 0)),
            pl.BlockSpec((N, 1), lambda b, j: (0, 0)),
            pl.BlockSpec((N, 1), lambda b, j: (0, 0)),
            pl.BlockSpec((N, 1), lambda b, j: (0, 0)),
            pl.BlockSpec((H, N), lambda b, j: (0, 0)),
            pl.BlockSpec((H, N), lambda b, j: (0, 0)),
        ],
        out_specs=pl.BlockSpec((Bb, H, Tt), lambda b, j: (b, 0, j)),
        out_shape=jax.ShapeDtypeStruct((B, H, T), jnp.float32),
        compiler_params=pltpu.CompilerParams(
            dimension_semantics=("parallel", "parallel")),
        interpret=interpret,
    )(x, p_lo, p_hi, e_lo, e_hi, ptab, etab)
